# Initial kernel scaffold; baseline (speedup 1.0000x reference)
import jax
import jax.numpy as jnp
from jax.experimental import pallas as pl


def kernel(x, edge_index, W1, b1, W2, b2, Wd, bd):
    raise NotImplementedError("write your pallas kernel here")


# speedup vs baseline: 17.8586x; 17.8586x over previous
import functools

import jax
import jax.numpy as jnp
from jax import lax
from jax.experimental import pallas as pl
from jax.experimental.pallas import tpu as pltpu
from jax.experimental.pallas import tpu_sc as plsc

N = 10000
F_IN = 128
H = 128
O = 64
E = 320000

ROW_BLOCK = 512
NPAD = 10240
NDUMMY = 64

NCORES = 2
NSUB = 16
NTILES = NCORES * NSUB
CHUNK = 128
CPT = 80
EDGES_PER_TILE = CPT * CHUNK
EPAD = NTILES * EDGES_PER_TILE
ROWS_PER_TILE = NPAD // NSUB

_MESH = plsc.VectorSubcoreMesh(core_axis_name="c", subcore_axis_name="s")


def _make_agg(feat):
    @functools.partial(
        pl.kernel,
        out_type=jax.ShapeDtypeStruct((NCORES, NPAD, feat), jnp.float32),
        mesh=_MESH,
        scratch_types=[
            pltpu.VMEM_SHARED((NPAD, feat), jnp.float32),
            pltpu.VMEM((CHUNK,), jnp.int32),
            pltpu.VMEM((CHUNK,), jnp.int32),
            pltpu.VMEM((CHUNK,), jnp.int32),
            pltpu.VMEM((CHUNK,), jnp.int32),
            pltpu.VMEM((CHUNK, feat), jnp.float32),
            pltpu.VMEM((CHUNK, feat), jnp.float32),
            pltpu.SemaphoreType.DMA,
            pltpu.SemaphoreType.DMA,
        ],
    )
    def agg(y_hbm, src_hbm, dst_hbm, zeros_hbm, out_hbm,
            acc_sh, sidx0, sidx1, didx0, didx1, rows0, rows1, sem0, sem1):
        c = lax.axis_index("c")
        s = lax.axis_index("s")
        base = (c * NSUB + s) * EDGES_PER_TILE
        r0 = s * ROWS_PER_TILE

        sidx = (sidx0, sidx1)
        didx = (didx0, didx1)
        rows = (rows0, rows1)
        sems = (sem0, sem1)

        def load(b, k):
            off = base + k * CHUNK
            pltpu.sync_copy(src_hbm.at[pl.ds(off, CHUNK)], sidx[b])
            pltpu.sync_copy(dst_hbm.at[pl.ds(off, CHUNK)], didx[b])
            pltpu.async_copy(y_hbm.at[sidx[b]], rows[b], sems[b])

        pltpu.sync_copy(zeros_hbm.at[pl.ds(r0, ROWS_PER_TILE)],
                        acc_sh.at[pl.ds(r0, ROWS_PER_TILE)])
        load(0, 0)
        load(1, 1)
        plsc.subcore_barrier()

        @pl.loop(0, CPT // 2)
        def _(i):
            for b in range(2):
                k = 2 * i + b
                pltpu.make_async_copy(y_hbm.at[sidx[b]], rows[b], sems[b]).wait()
                pltpu.sync_copy(rows[b], acc_sh.at[didx[b]], add=True)

                @pl.when(k + 2 < CPT)
                def _():
                    load(b, k + 2)

        plsc.subcore_barrier()
        pltpu.sync_copy(acc_sh.at[pl.ds(r0, ROWS_PER_TILE)],
                        out_hbm.at[c, pl.ds(r0, ROWS_PER_TILE)])

    return agg


_agg_h = _make_agg(H)


def _dis_of(degp_ref):
    deg = 1.0 + degp_ref[0, :, 0] + degp_ref[1, :, 0]
    return lax.rsqrt(deg)


_ONES_ROWS = 4096


def _mm1_body(x_ref, w_ref, o_ref):
    o_ref[...] = jnp.dot(x_ref[...], w_ref[...],
                         preferred_element_type=jnp.float32)


def _scale1_body(xw_ref, degp_ref, o_ref):
    dis = _dis_of(degp_ref)
    o_ref[...] = xw_ref[...] * dis[:, None]


def _k2_body(aggp_ref, y1_ref, degp_ref, b1_ref, w2_ref, o_ref):
    i = pl.program_id(0)
    dis = _dis_of(degp_ref)
    tot = aggp_ref[0] + aggp_ref[1] + y1_ref[...]
    h = jnp.maximum(tot * dis[:, None] + b1_ref[...][None, :], 0.0)
    y2 = jnp.dot(h, w2_ref[...], preferred_element_type=jnp.float32)
    y2 = y2 * dis[:, None]
    row = i * ROW_BLOCK + lax.broadcasted_iota(jnp.int32, (ROW_BLOCK, 1), 0)
    o_ref[...] = jnp.where(row < N, y2, 0.0)


def _k3_body(aggp_ref, y2_ref, degp_ref, b2_ref, wd_ref, bd_ref,
             emb_ref, rec_ref):
    dis = _dis_of(degp_ref)
    emb = (aggp_ref[0, :, :O] + aggp_ref[1, :, :O] + y2_ref[:, :O])
    emb = emb * dis[:, None] + b2_ref[...][None, :]
    emb_ref[...] = emb
    rec_ref[...] = jnp.dot(emb, wd_ref[...],
                           preferred_element_type=jnp.float32) + bd_ref[...][None, :]


_GRID = (NPAD // ROW_BLOCK,)


def _rows_spec(feat):
    return pl.BlockSpec((ROW_BLOCK, feat), lambda i: (i, 0))


def _degp_spec():
    return pl.BlockSpec((NCORES, ROW_BLOCK, H), lambda i: (0, i, 0))


def _aggp_spec(feat):
    return pl.BlockSpec((NCORES, ROW_BLOCK, feat), lambda i: (0, i, 0))


def _full_spec(shape):
    return pl.BlockSpec(shape, lambda i: tuple(0 for _ in shape))


def kernel(x, edge_index, W1, b1, W2, b2, Wd, bd):
    src = edge_index[0]
    dst = edge_index[1]
    pad_ids = N + (jnp.arange(EPAD - E, dtype=jnp.int32) % NDUMMY)
    src_p = jnp.concatenate([src, pad_ids])
    dst_p = jnp.concatenate([dst, pad_ids])
    x_pad = jnp.concatenate(
        [x, jnp.zeros((NPAD - N, F_IN), jnp.float32)], axis=0)
    zeros_h = jnp.zeros((NPAD, H), jnp.float32)
    w2_pad = jnp.concatenate([W2, jnp.zeros((H, H - O), jnp.float32)], axis=1)

    ones_tab = jnp.ones((_ONES_ROWS, H), jnp.float32)
    midx = jnp.arange(EPAD, dtype=jnp.int32) % _ONES_ROWS
    degp = _agg_h(ones_tab, midx, dst_p, zeros_h)

    xw1 = pl.pallas_call(
        _mm1_body,
        grid=_GRID,
        in_specs=[_rows_spec(F_IN), _full_spec((F_IN, H))],
        out_specs=_rows_spec(H),
        out_shape=jax.ShapeDtypeStruct((NPAD, H), jnp.float32),
    )(x_pad, W1)

    y1 = pl.pallas_call(
        _scale1_body,
        grid=_GRID,
        in_specs=[_rows_spec(H), _degp_spec()],
        out_specs=_rows_spec(H),
        out_shape=jax.ShapeDtypeStruct((NPAD, H), jnp.float32),
    )(xw1, degp)

    agg1 = _agg_h(y1, src_p, dst_p, zeros_h)

    y2 = pl.pallas_call(
        _k2_body,
        grid=_GRID,
        in_specs=[_aggp_spec(H), _rows_spec(H), _degp_spec(),
                  _full_spec((H,)), _full_spec((H, H))],
        out_specs=_rows_spec(H),
        out_shape=jax.ShapeDtypeStruct((NPAD, H), jnp.float32),
    )(agg1, y1, degp, b1, w2_pad)

    agg2 = _agg_h(y2, src_p, dst_p, zeros_h)

    emb_pad, rec_pad = pl.pallas_call(
        _k3_body,
        grid=_GRID,
        in_specs=[_aggp_spec(H), _rows_spec(H), _degp_spec(),
                  _full_spec((O,)), _full_spec((O, F_IN)),
                  _full_spec((F_IN,))],
        out_specs=[_rows_spec(O), _rows_spec(F_IN)],
        out_shape=[jax.ShapeDtypeStruct((NPAD, O), jnp.float32),
                   jax.ShapeDtypeStruct((NPAD, F_IN), jnp.float32)],
    )(agg2, y2, degp, b2, Wd, bd)

    return emb_pad[:N], rec_pad[:N]


# speedup vs baseline: 25.8819x; 1.4493x over previous
import functools

import jax
import jax.numpy as jnp
from jax import lax
from jax.experimental import pallas as pl
from jax.experimental.pallas import tpu as pltpu
from jax.experimental.pallas import tpu_sc as plsc

N = 10000
F_IN = 128
H = 128
O = 64
E = 320000

ROW_BLOCK = 512
NPAD = 10240
NDUMMY = 64

NCORES = 2
NSUB = 16
NTILES = NCORES * NSUB
CHUNK = 64
CPT = 160
EDGES_PER_TILE = CPT * CHUNK
EPAD = NTILES * EDGES_PER_TILE
ROWS_PER_TILE = NPAD // NSUB

_MESH = plsc.VectorSubcoreMesh(core_axis_name="c", subcore_axis_name="s")


NBUF = 4
CP2 = CPT // 2
IRINGP = 4


def _make_agg(feat):
    @functools.partial(
        pl.kernel,
        out_type=jax.ShapeDtypeStruct((NCORES, NPAD, feat), jnp.float32),
        mesh=_MESH,
        scratch_types=[
            pltpu.VMEM_SHARED((NPAD, feat), jnp.float32),
            pltpu.VMEM((CP2, 1, 2 * CHUNK), jnp.int32),
            pltpu.VMEM((IRINGP, 1, 2 * CHUNK), jnp.int32),
            pltpu.VMEM((NBUF * CHUNK, feat), jnp.float32),
        ] + [pltpu.SemaphoreType.DMA for _ in range(2 * NBUF + IRINGP + 1)],
    )
    def agg(y_hbm, src_hbm, dst_hbm, zeros_hbm, out_hbm,
            acc_sh, didx, sring, rows, *sems):
        gsem = sems[:NBUF]
        ssem = sems[NBUF:2 * NBUF]
        ism = sems[2 * NBUF:2 * NBUF + IRINGP]
        dsem = sems[2 * NBUF + IRINGP]
        c = lax.axis_index("c")
        s = lax.axis_index("s")
        prow0 = (c * NSUB + s) * CP2
        r0 = s * ROWS_PER_TILE

        def idx_load(slot, j):
            pltpu.async_copy(src_hbm.at[pl.ds(prow0 + j, 1)],
                             sring.at[pl.ds(slot, 1)], ism[slot])

        def idx_wait(slot):
            pltpu.make_async_copy(src_hbm.at[pl.ds(prow0, 1)],
                                  sring.at[pl.ds(slot, 1)], ism[slot]).wait()

        def gather(slot, half, sl):
            pltpu.async_copy(
                y_hbm.at[sring.at[slot, 0, pl.ds(half * CHUNK, CHUNK)]],
                rows.at[pl.ds(sl * CHUNK, CHUNK)], gsem[sl])

        def gather_wait(sl):
            pltpu.make_async_copy(
                y_hbm.at[sring.at[0, 0, pl.ds(0, CHUNK)]],
                rows.at[pl.ds(sl * CHUNK, CHUNK)], gsem[sl]).wait()

        def scatter(j, half, sl):
            pltpu.async_copy(
                rows.at[pl.ds(sl * CHUNK, CHUNK)],
                acc_sh.at[didx.at[j, 0, pl.ds(half * CHUNK, CHUNK)]],
                ssem[sl], add=True)

        def scatter_wait(sl):
            pltpu.make_async_copy(
                rows.at[pl.ds(sl * CHUNK, CHUNK)],
                acc_sh.at[didx.at[0, 0, pl.ds(0, CHUNK)]], ssem[sl]).wait()

        pltpu.async_copy(dst_hbm.at[pl.ds(prow0, CP2)], didx, dsem)
        for slot in range(IRINGP):
            idx_load(slot, slot)
        pltpu.sync_copy(zeros_hbm.at[pl.ds(r0, ROWS_PER_TILE)],
                        acc_sh.at[pl.ds(r0, ROWS_PER_TILE)])
        pltpu.make_async_copy(dst_hbm.at[pl.ds(prow0, CP2)], didx,
                              dsem).wait()
        idx_wait(0)
        gather(0, 0, 0)
        gather(0, 1, 1)
        idx_wait(1)
        gather(1, 0, 2)
        plsc.subcore_barrier()

        @pl.loop(0, CPT // 8)
        def _(i):
            for u in range(8):
                k = 8 * i + u
                sl = u % NBUF
                gather_wait(sl)
                scatter(k // 2, u % 2, sl)
                if u % 2 == 1:
                    @pl.when(k // 2 + IRINGP < CP2)
                    def _():
                        idx_load(u // 2, k // 2 + IRINGP)

                u3 = u + 3
                sl3 = u3 % NBUF
                slot3 = (u3 // 2) % IRINGP
                half3 = u3 % 2

                @pl.when(k + 3 < CPT)
                def _():
                    @pl.when(k >= 1)
                    def _():
                        scatter_wait(sl3)
                    if half3 == 0:
                        idx_wait(slot3)
                    gather(slot3, half3, sl3)

        for jj in range(NBUF):
            scatter_wait((CPT - NBUF + jj) % NBUF)

        plsc.subcore_barrier()
        pltpu.sync_copy(acc_sh.at[pl.ds(r0, ROWS_PER_TILE)],
                        out_hbm.at[c, pl.ds(r0, ROWS_PER_TILE)])

    return agg


_agg_h = _make_agg(H)


@functools.partial(
    pl.kernel,
    out_type=jax.ShapeDtypeStruct((NCORES, NPAD, H), jnp.float32),
    mesh=_MESH,
    scratch_types=[
        pltpu.VMEM_SHARED((NPAD, H), jnp.float32),
        pltpu.VMEM((CP2, 1, 2 * CHUNK), jnp.int32),
        pltpu.VMEM((CHUNK, H), jnp.float32),
    ] + [pltpu.SemaphoreType.DMA for _ in range(NBUF + 1)],
)
def _deg_kernel(dst_hbm, zeros_hbm, ones_hbm, out_hbm,
                acc_sh, didx, ones_v, *sems):
    ssem = sems[:NBUF]
    isem = sems[NBUF]
    c = lax.axis_index("c")
    s = lax.axis_index("s")
    prow0 = (c * NSUB + s) * CP2
    r0 = s * ROWS_PER_TILE

    pltpu.async_copy(dst_hbm.at[pl.ds(prow0, CP2)], didx, isem)
    pltpu.sync_copy(ones_hbm, ones_v)
    pltpu.sync_copy(zeros_hbm.at[pl.ds(r0, ROWS_PER_TILE)],
                    acc_sh.at[pl.ds(r0, ROWS_PER_TILE)])
    pltpu.make_async_copy(dst_hbm.at[pl.ds(prow0, CP2)], didx, isem).wait()
    plsc.subcore_barrier()

    @pl.loop(0, CPT // NBUF)
    def _(i):
        for b in range(NBUF):
            k = NBUF * i + b
            jj = k // 2
            half = b % 2

            @pl.when(k >= NBUF)
            def _():
                pltpu.make_async_copy(
                    ones_v, acc_sh.at[didx.at[0, 0, pl.ds(0, CHUNK)]],
                    ssem[b]).wait()

            pltpu.async_copy(
                ones_v, acc_sh.at[didx.at[jj, 0, pl.ds(half * CHUNK, CHUNK)]],
                ssem[b], add=True)

    for b in range(NBUF):
        pltpu.make_async_copy(
            ones_v, acc_sh.at[didx.at[0, 0, pl.ds(0, CHUNK)]], ssem[b]).wait()

    plsc.subcore_barrier()
    pltpu.sync_copy(acc_sh.at[pl.ds(r0, ROWS_PER_TILE)],
                    out_hbm.at[c, pl.ds(r0, ROWS_PER_TILE)])


def _dis_of(degp_ref):
    deg = 1.0 + degp_ref[0, :, 0] + degp_ref[1, :, 0]
    return lax.rsqrt(deg)


def _mm1_body(x_ref, w_ref, o_ref):
    o_ref[...] = jnp.dot(x_ref[...], w_ref[...],
                         preferred_element_type=jnp.float32)


def _scale1_body(xw_ref, degp_ref, o_ref):
    dis = _dis_of(degp_ref)
    o_ref[...] = xw_ref[...] * dis[:, None]


def _k2_body(aggp_ref, y1_ref, degp_ref, b1_ref, w2_ref, o_ref):
    i = pl.program_id(0)
    dis = _dis_of(degp_ref)
    tot = aggp_ref[0] + aggp_ref[1] + y1_ref[...]
    h = jnp.maximum(tot * dis[:, None] + b1_ref[...][None, :], 0.0)
    y2 = jnp.dot(h, w2_ref[...], preferred_element_type=jnp.float32)
    y2 = y2 * dis[:, None]
    row = i * ROW_BLOCK + lax.broadcasted_iota(jnp.int32, (ROW_BLOCK, 1), 0)
    o_ref[...] = jnp.where(row < N, y2, 0.0)


def _k3_body(aggp_ref, y2_ref, degp_ref, b2_ref, wd_ref, bd_ref,
             emb_ref, rec_ref):
    dis = _dis_of(degp_ref)
    emb = (aggp_ref[0, :, :O] + aggp_ref[1, :, :O] + y2_ref[:, :O])
    emb = emb * dis[:, None] + b2_ref[...][None, :]
    emb_ref[...] = emb
    rec_ref[...] = jnp.dot(emb, wd_ref[...],
                           preferred_element_type=jnp.float32) + bd_ref[...][None, :]


_GRID = (NPAD // ROW_BLOCK,)


def _rows_spec(feat):
    return pl.BlockSpec((ROW_BLOCK, feat), lambda i: (i, 0))


def _degp_spec():
    return pl.BlockSpec((NCORES, ROW_BLOCK, H), lambda i: (0, i, 0))


def _aggp_spec(feat):
    return pl.BlockSpec((NCORES, ROW_BLOCK, feat), lambda i: (0, i, 0))


def _full_spec(shape):
    return pl.BlockSpec(shape, lambda i: tuple(0 for _ in shape))


def kernel(x, edge_index, W1, b1, W2, b2, Wd, bd):
    src = edge_index[0]
    dst = edge_index[1]
    pad_ids = N + (jnp.arange(EPAD - E, dtype=jnp.int32) % NDUMMY)
    src_p = jnp.concatenate([src, pad_ids]).reshape(
        EPAD // (2 * CHUNK), 1, 2 * CHUNK)
    dst_p = jnp.concatenate([dst, pad_ids]).reshape(
        EPAD // (2 * CHUNK), 1, 2 * CHUNK)
    x_pad = jnp.concatenate(
        [x, jnp.zeros((NPAD - N, F_IN), jnp.float32)], axis=0)
    zeros_h = jnp.zeros((NPAD, H), jnp.float32)
    ones_ch = jnp.ones((CHUNK, H), jnp.float32)
    w2_pad = jnp.concatenate([W2, jnp.zeros((H, H - O), jnp.float32)], axis=1)

    degp = _deg_kernel(dst_p, zeros_h, ones_ch)

    xw1 = pl.pallas_call(
        _mm1_body,
        grid=_GRID,
        in_specs=[_rows_spec(F_IN), _full_spec((F_IN, H))],
        out_specs=_rows_spec(H),
        out_shape=jax.ShapeDtypeStruct((NPAD, H), jnp.float32),
    )(x_pad, W1)

    y1 = pl.pallas_call(
        _scale1_body,
        grid=_GRID,
        in_specs=[_rows_spec(H), _degp_spec()],
        out_specs=_rows_spec(H),
        out_shape=jax.ShapeDtypeStruct((NPAD, H), jnp.float32),
    )(xw1, degp)

    agg1 = _agg_h(y1, src_p, dst_p, zeros_h)

    y2 = pl.pallas_call(
        _k2_body,
        grid=_GRID,
        in_specs=[_aggp_spec(H), _rows_spec(H), _degp_spec(),
                  _full_spec((H,)), _full_spec((H, H))],
        out_specs=_rows_spec(H),
        out_shape=jax.ShapeDtypeStruct((NPAD, H), jnp.float32),
    )(agg1, y1, degp, b1, w2_pad)

    agg2 = _agg_h(y2, src_p, dst_p, zeros_h)

    emb_pad, rec_pad = pl.pallas_call(
        _k3_body,
        grid=_GRID,
        in_specs=[_aggp_spec(H), _rows_spec(H), _degp_spec(),
                  _full_spec((O,)), _full_spec((O, F_IN)),
                  _full_spec((F_IN,))],
        out_specs=[_rows_spec(O), _rows_spec(F_IN)],
        out_shape=[jax.ShapeDtypeStruct((NPAD, O), jnp.float32),
                   jax.ShapeDtypeStruct((NPAD, F_IN), jnp.float32)],
    )(agg2, y2, degp, b2, Wd, bd)

    return emb_pad[:N], rec_pad[:N]


# speedup vs baseline: 25.9290x; 1.0018x over previous
import functools

import jax
import jax.numpy as jnp
from jax import lax
from jax.experimental import pallas as pl
from jax.experimental.pallas import tpu as pltpu
from jax.experimental.pallas import tpu_sc as plsc

N = 10000
F_IN = 128
H = 128
O = 64
E = 320000

ROW_BLOCK = 512
NPAD = 10240
NDUMMY = 64

NCORES = 2
NSUB = 16
NTILES = NCORES * NSUB
CHUNK = 64
CPT = 160
EDGES_PER_TILE = CPT * CHUNK
EPAD = NTILES * EDGES_PER_TILE
ROWS_PER_TILE = NPAD // NSUB

_MESH = plsc.VectorSubcoreMesh(core_axis_name="c", subcore_axis_name="s")


NBUF = 4
CP2 = CPT // 2
IRINGP = 4


def _make_agg(feat):
    @functools.partial(
        pl.kernel,
        out_type=jax.ShapeDtypeStruct((NCORES, NPAD, feat), jnp.float32),
        mesh=_MESH,
        scratch_types=[
            pltpu.VMEM_SHARED((NPAD, feat), jnp.float32),
            pltpu.VMEM((CP2, 1, 2 * CHUNK), jnp.int32),
            pltpu.VMEM((IRINGP, 1, 2 * CHUNK), jnp.int32),
            pltpu.VMEM((NBUF * CHUNK, feat), jnp.float32),
        ] + [pltpu.SemaphoreType.DMA for _ in range(2 * NBUF + IRINGP + 1)],
    )
    def agg(y_hbm, src_hbm, dst_hbm, zeros_hbm, out_hbm,
            acc_sh, didx, sring, rows, *sems):
        gsem = sems[:NBUF]
        ssem = sems[NBUF:2 * NBUF]
        ism = sems[2 * NBUF:2 * NBUF + IRINGP]
        dsem = sems[2 * NBUF + IRINGP]
        c = lax.axis_index("c")
        s = lax.axis_index("s")
        prow0 = (c * NSUB + s) * CP2
        r0 = s * ROWS_PER_TILE

        def idx_load(slot, j):
            pltpu.async_copy(src_hbm.at[pl.ds(prow0 + j, 1)],
                             sring.at[pl.ds(slot, 1)], ism[slot])

        def idx_wait(slot):
            pltpu.make_async_copy(src_hbm.at[pl.ds(prow0, 1)],
                                  sring.at[pl.ds(slot, 1)], ism[slot]).wait()

        def gather(slot, half, sl):
            pltpu.async_copy(
                y_hbm.at[sring.at[slot, 0, pl.ds(half * CHUNK, CHUNK)]],
                rows.at[pl.ds(sl * CHUNK, CHUNK)], gsem[sl])

        def gather_wait(sl):
            pltpu.make_async_copy(
                y_hbm.at[sring.at[0, 0, pl.ds(0, CHUNK)]],
                rows.at[pl.ds(sl * CHUNK, CHUNK)], gsem[sl]).wait()

        def scatter(j, half, sl):
            pltpu.async_copy(
                rows.at[pl.ds(sl * CHUNK, CHUNK)],
                acc_sh.at[didx.at[j, 0, pl.ds(half * CHUNK, CHUNK)]],
                ssem[sl], add=True)

        def scatter_wait(sl):
            pltpu.make_async_copy(
                rows.at[pl.ds(sl * CHUNK, CHUNK)],
                acc_sh.at[didx.at[0, 0, pl.ds(0, CHUNK)]], ssem[sl]).wait()

        pltpu.async_copy(dst_hbm.at[pl.ds(prow0, CP2)], didx, dsem)
        for slot in range(IRINGP):
            idx_load(slot, slot)
        pltpu.sync_copy(zeros_hbm.at[pl.ds(r0, ROWS_PER_TILE)],
                        acc_sh.at[pl.ds(r0, ROWS_PER_TILE)])
        pltpu.make_async_copy(dst_hbm.at[pl.ds(prow0, CP2)], didx,
                              dsem).wait()
        idx_wait(0)
        gather(0, 0, 0)
        gather(0, 1, 1)
        idx_wait(1)
        gather(1, 0, 2)
        plsc.subcore_barrier()

        @pl.loop(0, CPT // 8)
        def _(i):
            for u in range(8):
                k = 8 * i + u
                sl = u % NBUF
                gather_wait(sl)
                scatter(k // 2, u % 2, sl)
                if u % 2 == 1:
                    @pl.when(k // 2 + IRINGP < CP2)
                    def _():
                        idx_load(u // 2, k // 2 + IRINGP)

                u3 = u + 3
                sl3 = u3 % NBUF
                slot3 = (u3 // 2) % IRINGP
                half3 = u3 % 2

                @pl.when(k + 3 < CPT)
                def _():
                    @pl.when(k >= 1)
                    def _():
                        scatter_wait(sl3)
                    if half3 == 0:
                        idx_wait(slot3)
                    gather(slot3, half3, sl3)

        for jj in range(NBUF):
            scatter_wait((CPT - NBUF + jj) % NBUF)

        plsc.subcore_barrier()
        pltpu.sync_copy(acc_sh.at[pl.ds(r0, ROWS_PER_TILE)],
                        out_hbm.at[c, pl.ds(r0, ROWS_PER_TILE)])

    return agg


_agg_h = _make_agg(H)


@functools.partial(
    pl.kernel,
    out_type=jax.ShapeDtypeStruct((NCORES, NPAD, H), jnp.float32),
    mesh=_MESH,
    scratch_types=[
        pltpu.VMEM_SHARED((NPAD, H), jnp.float32),
        pltpu.VMEM((CP2, 1, 2 * CHUNK), jnp.int32),
        pltpu.VMEM((CHUNK, H), jnp.float32),
    ] + [pltpu.SemaphoreType.DMA for _ in range(NBUF + 1)],
)
def _deg_kernel(dst_hbm, zeros_hbm, ones_hbm, out_hbm,
                acc_sh, didx, ones_v, *sems):
    ssem = sems[:NBUF]
    isem = sems[NBUF]
    c = lax.axis_index("c")
    s = lax.axis_index("s")
    prow0 = (c * NSUB + s) * CP2
    r0 = s * ROWS_PER_TILE

    pltpu.async_copy(dst_hbm.at[pl.ds(prow0, CP2)], didx, isem)
    pltpu.sync_copy(ones_hbm, ones_v)
    pltpu.sync_copy(zeros_hbm.at[pl.ds(r0, ROWS_PER_TILE)],
                    acc_sh.at[pl.ds(r0, ROWS_PER_TILE)])
    pltpu.make_async_copy(dst_hbm.at[pl.ds(prow0, CP2)], didx, isem).wait()
    plsc.subcore_barrier()

    @pl.loop(0, CPT // NBUF)
    def _(i):
        for b in range(NBUF):
            k = NBUF * i + b
            jj = k // 2
            half = b % 2

            @pl.when(k >= NBUF)
            def _():
                pltpu.make_async_copy(
                    ones_v, acc_sh.at[didx.at[0, 0, pl.ds(0, CHUNK)]],
                    ssem[b]).wait()

            pltpu.async_copy(
                ones_v, acc_sh.at[didx.at[jj, 0, pl.ds(half * CHUNK, CHUNK)]],
                ssem[b], add=True)

    for b in range(NBUF):
        pltpu.make_async_copy(
            ones_v, acc_sh.at[didx.at[0, 0, pl.ds(0, CHUNK)]], ssem[b]).wait()

    plsc.subcore_barrier()
    pltpu.sync_copy(acc_sh.at[pl.ds(r0, ROWS_PER_TILE)],
                    out_hbm.at[c, pl.ds(r0, ROWS_PER_TILE)])


def _dis_of(degp_ref):
    deg = 1.0 + degp_ref[0, :, 0] + degp_ref[1, :, 0]
    return lax.rsqrt(deg)


def _k1_body(x_ref, w_ref, degp_ref, o_ref):
    dis = _dis_of(degp_ref)
    xw = jnp.dot(x_ref[...], w_ref[...], preferred_element_type=jnp.float32)
    o_ref[...] = xw * dis[:, None]


def _k2_body(aggp_ref, y1_ref, degp_ref, b1_ref, w2_ref, o_ref):
    i = pl.program_id(0)
    dis = _dis_of(degp_ref)
    tot = aggp_ref[0] + aggp_ref[1] + y1_ref[...]
    h = jnp.maximum(tot * dis[:, None] + b1_ref[...][None, :], 0.0)
    y2 = jnp.dot(h, w2_ref[...], preferred_element_type=jnp.float32)
    y2 = y2 * dis[:, None]
    row = i * ROW_BLOCK + lax.broadcasted_iota(jnp.int32, (ROW_BLOCK, 1), 0)
    o_ref[...] = jnp.where(row < N, y2, 0.0)


def _k3_body(aggp_ref, y2_ref, degp_ref, b2_ref, wd_ref, bd_ref,
             emb_ref, rec_ref):
    dis = _dis_of(degp_ref)
    emb = (aggp_ref[0, :, :O] + aggp_ref[1, :, :O] + y2_ref[:, :O])
    emb = emb * dis[:, None] + b2_ref[...][None, :]
    emb_ref[...] = emb
    rec_ref[...] = jnp.dot(emb, wd_ref[...],
                           preferred_element_type=jnp.float32) + bd_ref[...][None, :]


_GRID = (NPAD // ROW_BLOCK,)


def _rows_spec(feat):
    return pl.BlockSpec((ROW_BLOCK, feat), lambda i: (i, 0))


def _degp_spec():
    return pl.BlockSpec((NCORES, ROW_BLOCK, H), lambda i: (0, i, 0))


def _aggp_spec(feat):
    return pl.BlockSpec((NCORES, ROW_BLOCK, feat), lambda i: (0, i, 0))


def _full_spec(shape):
    return pl.BlockSpec(shape, lambda i: tuple(0 for _ in shape))


def kernel(x, edge_index, W1, b1, W2, b2, Wd, bd):
    src = edge_index[0]
    dst = edge_index[1]
    pad_ids = N + (jnp.arange(EPAD - E, dtype=jnp.int32) % NDUMMY)
    src_p = jnp.concatenate([src, pad_ids]).reshape(
        EPAD // (2 * CHUNK), 1, 2 * CHUNK)
    dst_p = jnp.concatenate([dst, pad_ids]).reshape(
        EPAD // (2 * CHUNK), 1, 2 * CHUNK)
    x_pad = jnp.concatenate(
        [x, jnp.zeros((NPAD - N, F_IN), jnp.float32)], axis=0)
    zeros_h = jnp.zeros((NPAD, H), jnp.float32)
    ones_ch = jnp.ones((CHUNK, H), jnp.float32)
    w2_pad = jnp.concatenate([W2, jnp.zeros((H, H - O), jnp.float32)], axis=1)

    degp = _deg_kernel(dst_p, zeros_h, ones_ch)

    y1 = pl.pallas_call(
        _k1_body,
        grid=_GRID,
        in_specs=[_rows_spec(F_IN), _full_spec((F_IN, H)), _degp_spec()],
        out_specs=_rows_spec(H),
        out_shape=jax.ShapeDtypeStruct((NPAD, H), jnp.float32),
    )(x_pad, W1, degp)

    agg1 = _agg_h(y1, src_p, dst_p, zeros_h)

    y2 = pl.pallas_call(
        _k2_body,
        grid=_GRID,
        in_specs=[_aggp_spec(H), _rows_spec(H), _degp_spec(),
                  _full_spec((H,)), _full_spec((H, H))],
        out_specs=_rows_spec(H),
        out_shape=jax.ShapeDtypeStruct((NPAD, H), jnp.float32),
    )(agg1, y1, degp, b1, w2_pad)

    agg2 = _agg_h(y2, src_p, dst_p, zeros_h)

    emb_pad, rec_pad = pl.pallas_call(
        _k3_body,
        grid=_GRID,
        in_specs=[_aggp_spec(H), _rows_spec(H), _degp_spec(),
                  _full_spec((O,)), _full_spec((O, F_IN)),
                  _full_spec((F_IN,))],
        out_specs=[_rows_spec(O), _rows_spec(F_IN)],
        out_shape=[jax.ShapeDtypeStruct((NPAD, O), jnp.float32),
                   jax.ShapeDtypeStruct((NPAD, F_IN), jnp.float32)],
    )(agg2, y2, degp, b2, Wd, bd)

    return emb_pad[:N], rec_pad[:N]


# speedup vs baseline: 28.7295x; 1.1080x over previous
import functools

import jax
import jax.numpy as jnp
from jax import lax
from jax.experimental import pallas as pl
from jax.experimental.pallas import tpu as pltpu
from jax.experimental.pallas import tpu_sc as plsc

N = 10000
F_IN = 128
H = 128
O = 64
E = 320000

ROW_BLOCK = 1024
NPAD = 10240
NDUMMY = 64

NCORES = 2
NSUB = 16
NTILES = NCORES * NSUB
CHUNK = 64
CPT = 160
EDGES_PER_TILE = CPT * CHUNK
EPAD = NTILES * EDGES_PER_TILE
ROWS_PER_TILE = NPAD // NSUB

_MESH = plsc.VectorSubcoreMesh(core_axis_name="c", subcore_axis_name="s")


NBUF = 4
CP2 = CPT // 2
IRINGP = 4


def _make_agg(feat):
    @functools.partial(
        pl.kernel,
        out_type=jax.ShapeDtypeStruct((NCORES, NPAD, feat), jnp.float32),
        mesh=_MESH,
        scratch_types=[
            pltpu.VMEM_SHARED((NPAD, feat), jnp.float32),
            pltpu.VMEM((CP2, 2 * CHUNK), jnp.int32),
            pltpu.VMEM((IRINGP, 2 * CHUNK), jnp.int32),
            pltpu.VMEM((NBUF * CHUNK, feat), jnp.float32),
        ] + [pltpu.SemaphoreType.DMA for _ in range(2 * NBUF + IRINGP + 1)],
    )
    def agg(y_hbm, src_hbm, dst_hbm, out_hbm,
            acc_sh, didx, sring, rows, *sems):
        gsem = sems[:NBUF]
        ssem = sems[NBUF:2 * NBUF]
        ism = sems[2 * NBUF:2 * NBUF + IRINGP]
        dsem = sems[2 * NBUF + IRINGP]
        c = lax.axis_index("c")
        s = lax.axis_index("s")
        prow0 = (c * NSUB + s) * CP2
        r0 = s * ROWS_PER_TILE

        def idx_load(slot, j):
            pltpu.async_copy(src_hbm.at[pl.ds(prow0 + j, 1)],
                             sring.at[pl.ds(slot, 1)], ism[slot])

        def idx_wait(slot):
            pltpu.make_async_copy(src_hbm.at[pl.ds(prow0, 1)],
                                  sring.at[pl.ds(slot, 1)], ism[slot]).wait()

        def gather(slot, half, sl):
            pltpu.async_copy(
                y_hbm.at[sring.at[slot, pl.ds(half * CHUNK, CHUNK)]],
                rows.at[pl.ds(sl * CHUNK, CHUNK)], gsem[sl])

        def gather_wait(sl):
            pltpu.make_async_copy(
                y_hbm.at[sring.at[0, pl.ds(0, CHUNK)]],
                rows.at[pl.ds(sl * CHUNK, CHUNK)], gsem[sl]).wait()

        def scatter(j, half, sl):
            pltpu.async_copy(
                rows.at[pl.ds(sl * CHUNK, CHUNK)],
                acc_sh.at[didx.at[j, pl.ds(half * CHUNK, CHUNK)]],
                ssem[sl], add=True)

        def scatter_wait(sl):
            pltpu.make_async_copy(
                rows.at[pl.ds(sl * CHUNK, CHUNK)],
                acc_sh.at[didx.at[0, pl.ds(0, CHUNK)]], ssem[sl]).wait()

        pltpu.async_copy(dst_hbm.at[pl.ds(prow0, CP2)], didx, dsem)
        for slot in range(IRINGP):
            idx_load(slot, slot)
        zero16 = jnp.zeros((16,), jnp.float32)

        @pl.loop(0, CHUNK)
        def _(rr):
            @pl.loop(0, feat, step=16)
            def _(cc):
                rows[rr, pl.ds(cc, 16)] = zero16

        nzc = ROWS_PER_TILE // CHUNK
        for z in range(nzc):
            pltpu.sync_copy(rows.at[pl.ds(0, CHUNK)],
                            acc_sh.at[pl.ds(r0 + z * CHUNK, CHUNK)])
        pltpu.make_async_copy(dst_hbm.at[pl.ds(prow0, CP2)], didx,
                              dsem).wait()
        idx_wait(0)
        gather(0, 0, 0)
        gather(0, 1, 1)
        idx_wait(1)
        gather(1, 0, 2)
        plsc.subcore_barrier()

        @pl.loop(0, CPT // 8)
        def _(i):
            for u in range(8):
                k = 8 * i + u
                sl = u % NBUF
                gather_wait(sl)
                scatter(k // 2, u % 2, sl)
                if u % 2 == 1:
                    @pl.when(k // 2 + IRINGP < CP2)
                    def _():
                        idx_load(u // 2, k // 2 + IRINGP)

                u3 = u + 3
                sl3 = u3 % NBUF
                slot3 = (u3 // 2) % IRINGP
                half3 = u3 % 2

                @pl.when(k + 3 < CPT)
                def _():
                    @pl.when(k >= 1)
                    def _():
                        scatter_wait(sl3)
                    if half3 == 0:
                        idx_wait(slot3)
                    gather(slot3, half3, sl3)

        for jj in range(NBUF):
            scatter_wait((CPT - NBUF + jj) % NBUF)

        plsc.subcore_barrier()
        pltpu.sync_copy(acc_sh.at[pl.ds(r0, ROWS_PER_TILE)],
                        out_hbm.at[c, pl.ds(r0, ROWS_PER_TILE)])

    return agg


_agg_h = _make_agg(H)


@functools.partial(
    pl.kernel,
    out_type=jax.ShapeDtypeStruct((NCORES, NPAD, H), jnp.float32),
    mesh=_MESH,
    scratch_types=[
        pltpu.VMEM_SHARED((NPAD, H), jnp.float32),
        pltpu.VMEM((CP2, 2 * CHUNK), jnp.int32),
        pltpu.VMEM((CHUNK, H), jnp.float32),
    ] + [pltpu.SemaphoreType.DMA for _ in range(NBUF + 1)],
)
def _deg_kernel(dst_hbm, out_hbm, acc_sh, didx, ones_v, *sems):
    ssem = sems[:NBUF]
    isem = sems[NBUF]
    c = lax.axis_index("c")
    s = lax.axis_index("s")
    prow0 = (c * NSUB + s) * CP2
    r0 = s * ROWS_PER_TILE

    pltpu.async_copy(dst_hbm.at[pl.ds(prow0, CP2)], didx, isem)
    zero16 = jnp.zeros((16,), jnp.float32)
    one16 = jnp.ones((16,), jnp.float32)

    @pl.loop(0, CHUNK)
    def _(rr):
        @pl.loop(0, H, step=16)
        def _(cc):
            ones_v[rr, pl.ds(cc, 16)] = zero16

    for z in range(ROWS_PER_TILE // CHUNK):
        pltpu.sync_copy(ones_v.at[pl.ds(0, CHUNK)],
                        acc_sh.at[pl.ds(r0 + z * CHUNK, CHUNK)])

    @pl.loop(0, CHUNK)
    def _(rr):
        @pl.loop(0, H, step=16)
        def _(cc):
            ones_v[rr, pl.ds(cc, 16)] = one16

    pltpu.make_async_copy(dst_hbm.at[pl.ds(prow0, CP2)], didx, isem).wait()
    plsc.subcore_barrier()

    @pl.loop(0, CPT // NBUF)
    def _(i):
        for b in range(NBUF):
            k = NBUF * i + b
            jj = k // 2
            half = b % 2

            @pl.when(k >= NBUF)
            def _():
                pltpu.make_async_copy(
                    ones_v, acc_sh.at[didx.at[0, pl.ds(0, CHUNK)]],
                    ssem[b]).wait()

            pltpu.async_copy(
                ones_v, acc_sh.at[didx.at[jj, pl.ds(half * CHUNK, CHUNK)]],
                ssem[b], add=True)

    for b in range(NBUF):
        pltpu.make_async_copy(
            ones_v, acc_sh.at[didx.at[0, pl.ds(0, CHUNK)]], ssem[b]).wait()

    plsc.subcore_barrier()
    pltpu.sync_copy(acc_sh.at[pl.ds(r0, ROWS_PER_TILE)],
                    out_hbm.at[c, pl.ds(r0, ROWS_PER_TILE)])


def _dis_of(degp_ref):
    deg = 1.0 + degp_ref[0, :, 0] + degp_ref[1, :, 0]
    return lax.rsqrt(deg)


def _k1_body(x_ref, w_ref, degp_ref, o_ref, dis_ref):
    dis = _dis_of(degp_ref)
    xw = jnp.dot(x_ref[...], w_ref[...], preferred_element_type=jnp.float32)
    o_ref[...] = xw * dis[:, None]
    dis_ref[...] = jnp.broadcast_to(dis[:, None], (ROW_BLOCK, 16))


def _k2_body(aggp_ref, y1_ref, dis16_ref, b1_ref, w2_ref, o_ref):
    i = pl.program_id(0)
    dis = dis16_ref[:, 0]
    tot = aggp_ref[0] + aggp_ref[1] + y1_ref[...]
    h = jnp.maximum(tot * dis[:, None] + b1_ref[...][None, :], 0.0)
    y2 = jnp.dot(h, w2_ref[...], preferred_element_type=jnp.float32)
    y2 = y2 * dis[:, None]
    row = i * ROW_BLOCK + lax.broadcasted_iota(jnp.int32, (ROW_BLOCK, 1), 0)
    o_ref[...] = jnp.where(row < N, y2, 0.0)


def _k3_body(aggp_ref, y2_ref, dis16_ref, b2_ref, wd_ref, bd_ref,
             emb_ref, rec_ref):
    dis = dis16_ref[:, 0]
    emb = (aggp_ref[0, :, :O] + aggp_ref[1, :, :O] + y2_ref[:, :O])
    emb = emb * dis[:, None] + b2_ref[...][None, :]
    emb_ref[...] = emb
    rec_ref[...] = jnp.dot(emb, wd_ref[...],
                           preferred_element_type=jnp.float32) + bd_ref[...][None, :]


_GRID = (NPAD // ROW_BLOCK,)


def _rows_spec(feat):
    return pl.BlockSpec((ROW_BLOCK, feat), lambda i: (i, 0))


def _degp_spec():
    return pl.BlockSpec((NCORES, ROW_BLOCK, H), lambda i: (0, i, 0))


def _dis16_spec():
    return pl.BlockSpec((ROW_BLOCK, 16), lambda i: (i, 0))


def _aggp_spec(feat):
    return pl.BlockSpec((NCORES, ROW_BLOCK, feat), lambda i: (0, i, 0))


def _full_spec(shape):
    return pl.BlockSpec(shape, lambda i: tuple(0 for _ in shape))


def kernel(x, edge_index, W1, b1, W2, b2, Wd, bd):
    src = edge_index[0]
    dst = edge_index[1]
    pad_ids = N + (jnp.arange(EPAD - E, dtype=jnp.int32) % NDUMMY)
    src_p = jnp.concatenate([src, pad_ids]).reshape(
        EPAD // (2 * CHUNK), 2 * CHUNK)
    dst_p = jnp.concatenate([dst, pad_ids]).reshape(
        EPAD // (2 * CHUNK), 2 * CHUNK)
    x_pad = jnp.concatenate(
        [x, jnp.zeros((NPAD - N, F_IN), jnp.float32)], axis=0)
    w2_pad = jnp.concatenate([W2, jnp.zeros((H, H - O), jnp.float32)], axis=1)

    degp = _deg_kernel(dst_p)

    y1, dis16 = pl.pallas_call(
        _k1_body,
        grid=_GRID,
        in_specs=[_rows_spec(F_IN), _full_spec((F_IN, H)), _degp_spec()],
        out_specs=[_rows_spec(H), _dis16_spec()],
        out_shape=[jax.ShapeDtypeStruct((NPAD, H), jnp.float32),
                   jax.ShapeDtypeStruct((NPAD, 16), jnp.float32)],
    )(x_pad, W1, degp)

    agg1 = _agg_h(y1, src_p, dst_p)

    y2 = pl.pallas_call(
        _k2_body,
        grid=_GRID,
        in_specs=[_aggp_spec(H), _rows_spec(H), _dis16_spec(),
                  _full_spec((H,)), _full_spec((H, H))],
        out_specs=_rows_spec(H),
        out_shape=jax.ShapeDtypeStruct((NPAD, H), jnp.float32),
    )(agg1, y1, dis16, b1, w2_pad)

    agg2 = _agg_h(y2, src_p, dst_p)

    emb_pad, rec_pad = pl.pallas_call(
        _k3_body,
        grid=_GRID,
        in_specs=[_aggp_spec(H), _rows_spec(H), _dis16_spec(),
                  _full_spec((O,)), _full_spec((O, F_IN)),
                  _full_spec((F_IN,))],
        out_specs=[_rows_spec(O), _rows_spec(F_IN)],
        out_shape=[jax.ShapeDtypeStruct((N, O), jnp.float32),
                   jax.ShapeDtypeStruct((N, F_IN), jnp.float32)],
    )(agg2, y2, dis16, b2, Wd, bd)

    return emb_pad, rec_pad


# speedup vs baseline: 29.5123x; 1.0272x over previous
import functools

import jax
import jax.numpy as jnp
from jax import lax
from jax.experimental import pallas as pl
from jax.experimental.pallas import tpu as pltpu
from jax.experimental.pallas import tpu_sc as plsc

N = 10000
F_IN = 128
H = 128
O = 64
E = 320000

ROW_BLOCK = 1024
NPAD = 10240
NDUMMY = 64

NCORES = 2
NSUB = 16
NTILES = NCORES * NSUB
CHUNK = 64
CPT = 160
EDGES_PER_TILE = CPT * CHUNK
EPAD = NTILES * EDGES_PER_TILE
ROWS_PER_TILE = NPAD // NSUB

_MESH = plsc.VectorSubcoreMesh(core_axis_name="c", subcore_axis_name="s")


NBUF = 4
CP2 = CPT // 2
IRINGP = 4


def _make_agg(feat):
    @functools.partial(
        pl.kernel,
        out_type=jax.ShapeDtypeStruct((NCORES, NPAD, feat), jnp.float32),
        mesh=_MESH,
        scratch_types=[
            pltpu.VMEM_SHARED((NPAD, feat), jnp.float32),
            pltpu.VMEM((CP2, 2 * CHUNK), jnp.int32),
            pltpu.VMEM((IRINGP, 2 * CHUNK), jnp.int32),
            pltpu.VMEM((NBUF * CHUNK, feat), jnp.float32),
        ] + [pltpu.SemaphoreType.DMA for _ in range(2 * NBUF + IRINGP + 1)],
    )
    def agg(y_hbm, src_hbm, dst_hbm, out_hbm,
            acc_sh, didx, sring, rows, *sems):
        gsem = sems[:NBUF]
        ssem = sems[NBUF:2 * NBUF]
        ism = sems[2 * NBUF:2 * NBUF + IRINGP]
        dsem = sems[2 * NBUF + IRINGP]
        c = lax.axis_index("c")
        s = lax.axis_index("s")
        prow0 = (c * NSUB + s) * CP2
        r0 = s * ROWS_PER_TILE

        def idx_load(slot, j):
            pltpu.async_copy(src_hbm.at[pl.ds(prow0 + j, 1)],
                             sring.at[pl.ds(slot, 1)], ism[slot])

        def idx_wait(slot):
            pltpu.make_async_copy(src_hbm.at[pl.ds(prow0, 1)],
                                  sring.at[pl.ds(slot, 1)], ism[slot]).wait()

        def gather(slot, half, sl):
            pltpu.async_copy(
                y_hbm.at[sring.at[slot, pl.ds(half * CHUNK, CHUNK)]],
                rows.at[pl.ds(sl * CHUNK, CHUNK)], gsem[sl])

        def gather_wait(sl):
            pltpu.make_async_copy(
                y_hbm.at[sring.at[0, pl.ds(0, CHUNK)]],
                rows.at[pl.ds(sl * CHUNK, CHUNK)], gsem[sl]).wait()

        def scatter(j, half, sl):
            pltpu.async_copy(
                rows.at[pl.ds(sl * CHUNK, CHUNK)],
                acc_sh.at[didx.at[j, pl.ds(half * CHUNK, CHUNK)]],
                ssem[sl], add=True)

        def scatter_wait(sl):
            pltpu.make_async_copy(
                rows.at[pl.ds(sl * CHUNK, CHUNK)],
                acc_sh.at[didx.at[0, pl.ds(0, CHUNK)]], ssem[sl]).wait()

        pltpu.async_copy(dst_hbm.at[pl.ds(prow0, CP2)], didx, dsem)
        for slot in range(IRINGP):
            idx_load(slot, slot)
        zero16 = jnp.zeros((16,), jnp.float32)

        @pl.loop(0, CHUNK)
        def _(rr):
            @pl.loop(0, feat, step=16)
            def _(cc):
                rows[rr, pl.ds(cc, 16)] = zero16

        nzc = ROWS_PER_TILE // CHUNK
        for z in range(nzc):
            pltpu.sync_copy(rows.at[pl.ds(0, CHUNK)],
                            acc_sh.at[pl.ds(r0 + z * CHUNK, CHUNK)])
        pltpu.make_async_copy(dst_hbm.at[pl.ds(prow0, CP2)], didx,
                              dsem).wait()
        idx_wait(0)
        gather(0, 0, 0)
        gather(0, 1, 1)
        idx_wait(1)
        gather(1, 0, 2)
        plsc.subcore_barrier()

        @pl.loop(0, CPT // 8)
        def _(i):
            for u in range(8):
                k = 8 * i + u
                sl = u % NBUF
                gather_wait(sl)
                scatter(k // 2, u % 2, sl)
                if u % 2 == 1:
                    @pl.when(k // 2 + IRINGP < CP2)
                    def _():
                        idx_load(u // 2, k // 2 + IRINGP)

                u3 = u + 3
                sl3 = u3 % NBUF
                slot3 = (u3 // 2) % IRINGP
                half3 = u3 % 2

                @pl.when(k + 3 < CPT)
                def _():
                    @pl.when(k >= 1)
                    def _():
                        scatter_wait(sl3)
                    if half3 == 0:
                        idx_wait(slot3)
                    gather(slot3, half3, sl3)

        for jj in range(NBUF):
            scatter_wait((CPT - NBUF + jj) % NBUF)

        plsc.subcore_barrier()
        pltpu.sync_copy(acc_sh.at[pl.ds(r0, ROWS_PER_TILE)],
                        out_hbm.at[c, pl.ds(r0, ROWS_PER_TILE)])

    return agg


_agg_h = _make_agg(H)


@functools.partial(
    pl.kernel,
    out_type=jax.ShapeDtypeStruct((NCORES, NPAD, H), jnp.float32),
    mesh=_MESH,
    scratch_types=[
        pltpu.VMEM_SHARED((NPAD, H), jnp.float32),
        pltpu.VMEM((CP2, 2 * CHUNK), jnp.int32),
        pltpu.VMEM((CHUNK, H), jnp.float32),
    ] + [pltpu.SemaphoreType.DMA for _ in range(NBUF + 1)],
)
def _deg_kernel(dst_hbm, out_hbm, acc_sh, didx, ones_v, *sems):
    ssem = sems[:NBUF]
    isem = sems[NBUF]
    c = lax.axis_index("c")
    s = lax.axis_index("s")
    prow0 = (c * NSUB + s) * CP2
    r0 = s * ROWS_PER_TILE

    pltpu.async_copy(dst_hbm.at[pl.ds(prow0, CP2)], didx, isem)
    zero16 = jnp.zeros((16,), jnp.float32)
    one16 = jnp.ones((16,), jnp.float32)

    @pl.loop(0, CHUNK)
    def _(rr):
        @pl.loop(0, H, step=16)
        def _(cc):
            ones_v[rr, pl.ds(cc, 16)] = zero16

    for z in range(ROWS_PER_TILE // CHUNK):
        pltpu.sync_copy(ones_v.at[pl.ds(0, CHUNK)],
                        acc_sh.at[pl.ds(r0 + z * CHUNK, CHUNK)])

    @pl.loop(0, CHUNK)
    def _(rr):
        @pl.loop(0, H, step=16)
        def _(cc):
            ones_v[rr, pl.ds(cc, 16)] = one16

    pltpu.make_async_copy(dst_hbm.at[pl.ds(prow0, CP2)], didx, isem).wait()
    plsc.subcore_barrier()

    @pl.loop(0, CPT // NBUF)
    def _(i):
        for b in range(NBUF):
            k = NBUF * i + b
            jj = k // 2
            half = b % 2

            @pl.when(k >= NBUF)
            def _():
                pltpu.make_async_copy(
                    ones_v, acc_sh.at[didx.at[0, pl.ds(0, CHUNK)]],
                    ssem[b]).wait()

            pltpu.async_copy(
                ones_v, acc_sh.at[didx.at[jj, pl.ds(half * CHUNK, CHUNK)]],
                ssem[b], add=True)

    for b in range(NBUF):
        pltpu.make_async_copy(
            ones_v, acc_sh.at[didx.at[0, pl.ds(0, CHUNK)]], ssem[b]).wait()

    plsc.subcore_barrier()
    pltpu.sync_copy(acc_sh.at[pl.ds(r0, ROWS_PER_TILE)],
                    out_hbm.at[c, pl.ds(r0, ROWS_PER_TILE)])


def _dis_of(degp_ref):
    deg = 1.0 + degp_ref[0, :, 0] + degp_ref[1, :, 0]
    return lax.rsqrt(deg)


_PREP_COLS = 32768


def _prep_body(e_ref, src_ref, dst_ref):
    i = pl.program_id(0)
    rows = _PREP_COLS // 128
    r = i * rows + lax.broadcasted_iota(jnp.int32, (rows, 128), 0)
    lane = lax.broadcasted_iota(jnp.int32, (rows, 128), 1)
    eidx = r * 128 + lane
    pad_val = N + (eidx % NDUMMY)
    ok = eidx < E
    src_ref[...] = jnp.where(ok, e_ref[0, :].reshape(rows, 128), pad_val)
    dst_ref[...] = jnp.where(ok, e_ref[1, :].reshape(rows, 128), pad_val)


def _prep_edges(edge_index):
    rows = _PREP_COLS // 128
    grid = (EPAD // _PREP_COLS,)
    return pl.pallas_call(
        _prep_body,
        grid=grid,
        in_specs=[pl.BlockSpec((2, _PREP_COLS), lambda i: (0, i))],
        out_specs=[pl.BlockSpec((rows, 128), lambda i: (i, 0)),
                   pl.BlockSpec((rows, 128), lambda i: (i, 0))],
        out_shape=[jax.ShapeDtypeStruct((EPAD // 128, 128), jnp.int32),
                   jax.ShapeDtypeStruct((EPAD // 128, 128), jnp.int32)],
    )(edge_index)


def _k1_body(x_ref, w_ref, degp_ref, o_ref, dis_ref):
    dis = _dis_of(degp_ref)
    xw = jnp.dot(x_ref[...], w_ref[...], preferred_element_type=jnp.float32)
    o_ref[...] = xw * dis[:, None]
    dis_ref[...] = jnp.broadcast_to(dis[:, None], (ROW_BLOCK, 16))


def _k2_body(aggp_ref, y1_ref, dis16_ref, b1_ref, w2_ref, o_ref):
    i = pl.program_id(0)
    dis = dis16_ref[:, 0]
    tot = aggp_ref[0] + aggp_ref[1] + y1_ref[...]
    h = jnp.maximum(tot * dis[:, None] + b1_ref[...][None, :], 0.0)
    y2 = jnp.dot(h, w2_ref[...], preferred_element_type=jnp.float32)
    y2 = y2 * dis[:, None]
    row = i * ROW_BLOCK + lax.broadcasted_iota(jnp.int32, (ROW_BLOCK, 1), 0)
    o_ref[...] = jnp.where(row < N, y2, 0.0)


def _k3_body(aggp_ref, y2_ref, dis16_ref, b2_ref, wd_ref, bd_ref,
             emb_ref, rec_ref):
    dis = dis16_ref[:, 0]
    emb = (aggp_ref[0, :, :O] + aggp_ref[1, :, :O] + y2_ref[:, :O])
    emb = emb * dis[:, None] + b2_ref[...][None, :]
    emb_ref[...] = emb
    rec_ref[...] = jnp.dot(emb, wd_ref[...],
                           preferred_element_type=jnp.float32) + bd_ref[...][None, :]


_GRID = (NPAD // ROW_BLOCK,)


def _rows_spec(feat):
    return pl.BlockSpec((ROW_BLOCK, feat), lambda i: (i, 0))


def _degp_spec():
    return pl.BlockSpec((NCORES, ROW_BLOCK, H), lambda i: (0, i, 0))


def _dis16_spec():
    return pl.BlockSpec((ROW_BLOCK, 16), lambda i: (i, 0))


def _aggp_spec(feat):
    return pl.BlockSpec((NCORES, ROW_BLOCK, feat), lambda i: (0, i, 0))


def _full_spec(shape):
    return pl.BlockSpec(shape, lambda i: tuple(0 for _ in shape))


def kernel(x, edge_index, W1, b1, W2, b2, Wd, bd):
    src_p, dst_p = _prep_edges(edge_index)
    x_pad = jnp.concatenate(
        [x, jnp.zeros((NPAD - N, F_IN), jnp.float32)], axis=0)
    w2_pad = jnp.concatenate([W2, jnp.zeros((H, H - O), jnp.float32)], axis=1)

    degp = _deg_kernel(dst_p)

    y1, dis16 = pl.pallas_call(
        _k1_body,
        grid=_GRID,
        in_specs=[_rows_spec(F_IN), _full_spec((F_IN, H)), _degp_spec()],
        out_specs=[_rows_spec(H), _dis16_spec()],
        out_shape=[jax.ShapeDtypeStruct((NPAD, H), jnp.float32),
                   jax.ShapeDtypeStruct((NPAD, 16), jnp.float32)],
    )(x_pad, W1, degp)

    agg1 = _agg_h(y1, src_p, dst_p)

    y2 = pl.pallas_call(
        _k2_body,
        grid=_GRID,
        in_specs=[_aggp_spec(H), _rows_spec(H), _dis16_spec(),
                  _full_spec((H,)), _full_spec((H, H))],
        out_specs=_rows_spec(H),
        out_shape=jax.ShapeDtypeStruct((NPAD, H), jnp.float32),
    )(agg1, y1, dis16, b1, w2_pad)

    agg2 = _agg_h(y2, src_p, dst_p)

    emb_pad, rec_pad = pl.pallas_call(
        _k3_body,
        grid=_GRID,
        in_specs=[_aggp_spec(H), _rows_spec(H), _dis16_spec(),
                  _full_spec((O,)), _full_spec((O, F_IN)),
                  _full_spec((F_IN,))],
        out_specs=[_rows_spec(O), _rows_spec(F_IN)],
        out_shape=[jax.ShapeDtypeStruct((N, O), jnp.float32),
                   jax.ShapeDtypeStruct((N, F_IN), jnp.float32)],
    )(agg2, y2, dis16, b2, Wd, bd)

    return emb_pad, rec_pad
